# Initial kernel scaffold; baseline (speedup 1.0000x reference)
import jax
import jax.numpy as jnp
from jax.experimental import pallas as pl


def kernel(x, table):
    raise NotImplementedError("write your pallas kernel here")


# speedup vs baseline: 1.1028x; 1.1028x over previous
import functools

import jax
import jax.numpy as jnp
from jax import lax
from jax.experimental import pallas as pl
from jax.experimental.pallas import tpu as pltpu
from jax.experimental.pallas import tpu_sc as plsc


def _make_sc_gather(B, D):
    info = plsc.get_sparse_core_info()
    NC, NS = info.num_cores, info.num_subcores
    NW = NC * NS
    assert B % NW == 0
    b_per_w = B // NW

    CHUNK = 128
    GROUP = 8
    ROWS = CHUNK * GROUP
    assert b_per_w % ROWS == 0
    n_groups = b_per_w // ROWS
    n_chunks = b_per_w // CHUNK

    mesh = plsc.VectorSubcoreMesh(core_axis_name="c", subcore_axis_name="s")

    @functools.partial(
        pl.kernel,
        mesh=mesh,
        compiler_params=pltpu.CompilerParams(use_tc_tiling_on_sc=False),
        out_type=jax.ShapeDtypeStruct((B, D), jnp.float32),
        scratch_types=[
            pltpu.VMEM((n_chunks, CHUNK), jnp.int32),
            pltpu.VMEM((ROWS, D), jnp.float32),
            pltpu.SemaphoreType.DMA,
        ],
    )
    def gather_kernel(idx_hbm, table_hbm, out_hbm, idx_v, rows_v, sem):
        wid = lax.axis_index("s") * NC + lax.axis_index("c")
        base = wid * b_per_w

        pltpu.sync_copy(idx_hbm.at[wid], idx_v)

        def group_body(g, carry):
            handles = []
            for j in range(GROUP):
                handles.append(
                    pltpu.async_copy(
                        table_hbm.at[idx_v.at[g * GROUP + j]],
                        rows_v.at[pl.ds(j * CHUNK, CHUNK)],
                        sem,
                    )
                )
            for h in handles:
                h.wait()
            pltpu.sync_copy(rows_v, out_hbm.at[pl.ds(base + g * ROWS, ROWS)])
            return carry

        lax.fori_loop(0, n_groups, group_body, 0)

    def run(idx_flat, table):
        idx3 = idx_flat.reshape(NW, n_chunks, CHUNK)
        return gather_kernel(idx3, table)

    return run


def kernel(x, table):
    B, L = x.shape
    V, D = table.shape
    run = _make_sc_gather(B * L, D)
    out = run(x.reshape(-1).astype(jnp.int32), table)
    return out.reshape(B, L, D)


# speedup vs baseline: 1.1133x; 1.0095x over previous
import functools

import jax
import jax.numpy as jnp
from jax import lax
from jax.experimental import pallas as pl
from jax.experimental.pallas import tpu as pltpu
from jax.experimental.pallas import tpu_sc as plsc


def _make_sc_gather(B, D):
    info = plsc.get_sparse_core_info()
    NC, NS = info.num_cores, info.num_subcores
    NW = NC * NS
    assert B % NW == 0
    b_per_w = B // NW

    CHUNK = 128
    GROUP = 8
    ROWS = CHUNK * GROUP
    NBUF = 3
    assert b_per_w % ROWS == 0
    n_groups = b_per_w // ROWS
    n_chunks = b_per_w // CHUNK
    assert n_groups > NBUF

    mesh = plsc.VectorSubcoreMesh(core_axis_name="c", subcore_axis_name="s")

    @functools.partial(
        pl.kernel,
        mesh=mesh,
        compiler_params=pltpu.CompilerParams(use_tc_tiling_on_sc=False),
        out_type=jax.ShapeDtypeStruct((B, D), jnp.float32),
        scratch_types=[
            pltpu.VMEM((n_chunks, CHUNK), jnp.int32),
            pltpu.VMEM((NBUF, ROWS, D), jnp.float32),
            pltpu.SemaphoreType.DMA((NBUF,)),
            pltpu.SemaphoreType.DMA((NBUF,)),
        ],
    )
    def gather_kernel(idx_hbm, table_hbm, out_hbm, idx_v, rows_v, gsem, ssem):
        wid = lax.axis_index("s") * NC + lax.axis_index("c")
        base = wid * b_per_w

        pltpu.sync_copy(idx_hbm.at[wid], idx_v)

        def issue_gathers(g, b):
            for j in range(GROUP):
                pltpu.async_copy(
                    table_hbm.at[idx_v.at[g * GROUP + j]],
                    rows_v.at[b, pl.ds(j * CHUNK, CHUNK)],
                    gsem.at[b],
                )

        def wait_bytes(dst, sem):
            pltpu.make_async_copy(out_hbm.at[pl.ds(0, dst.shape[0])], dst, sem).wait()

        for p in range(NBUF - 1):
            issue_gathers(p, p)

        def group_body(g, carry):
            b = g % NBUF
            wait_bytes(rows_v.at[b], gsem.at[b])
            pltpu.async_copy(
                rows_v.at[b],
                out_hbm.at[pl.ds(base + g * ROWS, ROWS)],
                ssem.at[b],
            )
            gn = g + NBUF - 1
            bn = gn % NBUF

            @pl.when(jnp.logical_and(gn < n_groups, g >= 1))
            def _():
                wait_bytes(rows_v.at[bn], ssem.at[bn])

            @pl.when(gn < n_groups)
            def _():
                issue_gathers(gn, bn)

            return carry

        lax.fori_loop(0, n_groups, group_body, 0)

        for t in range(n_groups - NBUF, n_groups):
            wait_bytes(rows_v.at[t % NBUF], ssem.at[t % NBUF])

    def run(idx_flat, table):
        idx3 = idx_flat.reshape(NW, n_chunks, CHUNK)
        return gather_kernel(idx3, table)

    return run


def kernel(x, table):
    B, L = x.shape
    V, D = table.shape
    run = _make_sc_gather(B * L, D)
    out = run(x.reshape(-1).astype(jnp.int32), table)
    return out.reshape(B, L, D)


# speedup vs baseline: 1.5675x; 1.4080x over previous
import functools

import jax
import jax.numpy as jnp
from jax import lax
from jax.experimental import pallas as pl
from jax.experimental.pallas import tpu as pltpu
from jax.experimental.pallas import tpu_sc as plsc

BATCH = 16384
HIST = 50
LPAD = 64
EMB = 32
ROW = 128


def _tca(table_t):
    V = table_t.shape[1]
    BLK = 8192
    grid = (pl.cdiv(V, BLK),)

    def body(tin, tout):
        tout[:, :EMB] = tin[...].T

    return pl.pallas_call(
        body,
        grid=grid,
        in_specs=[pl.BlockSpec((EMB, BLK), lambda i: (0, i))],
        out_specs=pl.BlockSpec((BLK, ROW), lambda i: (i, 0)),
        out_shape=jax.ShapeDtypeStruct((V, ROW), jnp.float32),
    )(table_t)


def _tcc(outp3):
    BB = 512
    grid = (BATCH // BB,)

    def body(tin, tout):
        t = tin[...].reshape(BB, HIST * ROW).T
        tout[...] = t.reshape(HIST, ROW, BB)[:, :EMB, :]

    return pl.pallas_call(
        body,
        grid=grid,
        in_specs=[pl.BlockSpec((BB, HIST, ROW), lambda i: (i, 0, 0))],
        out_specs=pl.BlockSpec((HIST, EMB, BB), lambda i: (0, 0, i)),
        out_shape=jax.ShapeDtypeStruct((HIST, EMB, BATCH), jnp.float32),
    )(outp3)


def _make_scb(V):
    info = plsc.get_sparse_core_info()
    NC, NS = info.num_cores, info.num_subcores
    NW = NC * NS
    b_per_w = BATCH // NW
    GB = 4
    NBUF = 2
    n_groups = b_per_w // GB

    mesh = plsc.VectorSubcoreMesh(core_axis_name="c", subcore_axis_name="s")

    @functools.partial(
        pl.kernel,
        mesh=mesh,
        compiler_params=pltpu.CompilerParams(needs_layout_passes=False),
        out_type=jax.ShapeDtypeStruct((BATCH * HIST, ROW), jnp.float32),
        scratch_types=[
            pltpu.VMEM((HIST, b_per_w), jnp.int32),
            pltpu.VMEM((b_per_w * LPAD,), jnp.int32),
            pltpu.VMEM((NBUF, GB * HIST, ROW), jnp.float32),
            pltpu.SemaphoreType.DMA((NBUF,)),
            pltpu.SemaphoreType.DMA((NBUF,)),
        ],
    )
    def scb(xt_hbm, tab_hbm, outp_hbm, xvt, xb, rows_v, gsem, ssem):
        cid = lax.axis_index("c")
        sid = lax.axis_index("s")
        wid = cid * NS + sid
        base = wid * b_per_w

        pltpu.sync_copy(xt_hbm.at[:, pl.ds(base, b_per_w)], xvt)

        lanes = lax.iota(jnp.int32, 16)

        def tr_body(b, carry):
            for l0 in range(0, LPAD, 16):
                rows = jnp.minimum(lanes + l0, HIST - 1)
                vals = plsc.load_gather(
                    xvt, [rows, jnp.full((16,), b, jnp.int32)])
                xb[pl.ds(b * LPAD + l0, 16)] = vals
            return carry

        lax.fori_loop(0, b_per_w, tr_body, 0)

        def issue_gathers(g, b):
            for j in range(GB):
                pltpu.async_copy(
                    tab_hbm.at[xb.at[pl.ds((g * GB + j) * LPAD, HIST)]],
                    rows_v.at[b, pl.ds(j * HIST, HIST)],
                    gsem.at[b],
                )

        def wait_gathers(g, b):
            for j in range(GB):
                pltpu.make_async_copy(
                    tab_hbm.at[xb.at[pl.ds((g * GB + j) * LPAD, HIST)]],
                    rows_v.at[b, pl.ds(j * HIST, HIST)],
                    gsem.at[b],
                ).wait()

        def wait_store(b):
            pltpu.make_async_copy(
                rows_v.at[b],
                outp_hbm.at[pl.ds(0, GB * HIST)],
                ssem.at[b],
            ).wait()

        issue_gathers(0, 0)

        def group_body(g, carry):
            b = g % NBUF
            wait_gathers(g, b)
            pltpu.async_copy(
                rows_v.at[b],
                outp_hbm.at[pl.ds((base + g * GB) * HIST, GB * HIST)],
                ssem.at[b],
            )
            gn = g + 1
            bn = gn % NBUF

            @pl.when(jnp.logical_and(gn < n_groups, g >= 1))
            def _():
                wait_store(bn)

            @pl.when(gn < n_groups)
            def _():
                issue_gathers(gn, bn)

            return carry

        lax.fori_loop(0, n_groups, group_body, 0)

        for t in range(n_groups - NBUF, n_groups):
            wait_store(t % NBUF)

    return scb


def kernel(x, table):
    V, D = table.shape
    xt = x.astype(jnp.int32).T
    tab_p = _tca(table.T)
    out_p = _make_scb(V)(xt, tab_p)
    out_t = _tcc(out_p.reshape(BATCH, HIST, ROW))
    return out_t.transpose(2, 0, 1)


# speedup vs baseline: 2.3314x; 1.4873x over previous
import functools

import jax
import jax.numpy as jnp
from jax import lax
from jax.experimental import pallas as pl
from jax.experimental.pallas import tpu as pltpu
from jax.experimental.pallas import tpu_sc as plsc

BATCH = 16384
HIST = 50
LPAD = 64
EMB = 32
ROW = 128


def _tca(table_t):
    V = table_t.shape[1]
    BLK = 8192
    grid = (pl.cdiv(V, BLK),)

    def body(tin, tout):
        tout[:, :EMB] = tin[...].T

    return pl.pallas_call(
        body,
        grid=grid,
        in_specs=[pl.BlockSpec((EMB, BLK), lambda i: (0, i))],
        out_specs=pl.BlockSpec((BLK, ROW), lambda i: (i, 0)),
        out_shape=jax.ShapeDtypeStruct((V, ROW), jnp.float32),
    )(table_t)


def _tcc(outp):
    BB = 512
    grid = (BATCH // BB,)

    def body(tin, tout):
        t = tin[...].reshape(BB, HIST * ROW).T
        tout[...] = t.reshape(HIST, ROW, BB)[:, :EMB, :]

    return pl.pallas_call(
        body,
        grid=grid,
        in_specs=[pl.BlockSpec((BB * HIST, ROW), lambda i: (i, 0))],
        out_specs=pl.BlockSpec((HIST, EMB, BB), lambda i: (0, 0, i)),
        out_shape=jax.ShapeDtypeStruct((HIST, EMB, BATCH), jnp.float32),
    )(outp)


def _make_scb(V):
    info = plsc.get_sparse_core_info()
    NC, NS = info.num_cores, info.num_subcores
    NW = NC * NS
    b_per_w = BATCH // NW
    GB = 4
    NBUF = 2
    n_groups = b_per_w // GB

    mesh = plsc.VectorSubcoreMesh(core_axis_name="c", subcore_axis_name="s")

    @functools.partial(
        pl.kernel,
        mesh=mesh,
        compiler_params=pltpu.CompilerParams(needs_layout_passes=False),
        out_type=jax.ShapeDtypeStruct((BATCH * HIST, ROW), jnp.float32),
        scratch_types=[
            pltpu.VMEM((HIST, b_per_w), jnp.int32),
            pltpu.VMEM((b_per_w * LPAD,), jnp.int32),
            pltpu.VMEM((NBUF, GB * HIST, ROW), jnp.float32),
            pltpu.SemaphoreType.DMA((NBUF,)),
            pltpu.SemaphoreType.DMA((NBUF,)),
        ],
    )
    def scb(xt_hbm, tab_hbm, outp_hbm, xvt, xb, rows_v, gsem, ssem):
        cid = lax.axis_index("c")
        sid = lax.axis_index("s")
        wid = cid * NS + sid
        base = wid * b_per_w

        pltpu.sync_copy(xt_hbm.at[:, pl.ds(base, b_per_w)], xvt)

        lanes = lax.iota(jnp.int32, 16)

        def tr_body(b, carry):
            for l0 in range(0, LPAD, 16):
                rows = jnp.minimum(lanes + l0, HIST - 1)
                vals = plsc.load_gather(
                    xvt, [rows, jnp.full((16,), b, jnp.int32)])
                xb[pl.ds(b * LPAD + l0, 16)] = vals
            return carry

        lax.fori_loop(0, b_per_w, tr_body, 0)

        def issue_gathers(g, b):
            for j in range(GB):
                pltpu.async_copy(
                    tab_hbm.at[xb.at[pl.ds((g * GB + j) * LPAD, HIST)]],
                    rows_v.at[b, pl.ds(j * HIST, HIST)],
                    gsem.at[b],
                )

        def wait_gathers(g, b):
            for j in range(GB):
                pltpu.make_async_copy(
                    tab_hbm.at[xb.at[pl.ds((g * GB + j) * LPAD, HIST)]],
                    rows_v.at[b, pl.ds(j * HIST, HIST)],
                    gsem.at[b],
                ).wait()

        def wait_store(b):
            pltpu.make_async_copy(
                rows_v.at[b],
                outp_hbm.at[pl.ds(0, GB * HIST)],
                ssem.at[b],
            ).wait()

        issue_gathers(0, 0)

        def group_body(g, carry):
            b = g % NBUF
            wait_gathers(g, b)
            pltpu.async_copy(
                rows_v.at[b],
                outp_hbm.at[pl.ds((base + g * GB) * HIST, GB * HIST)],
                ssem.at[b],
            )
            gn = g + 1
            bn = gn % NBUF

            @pl.when(jnp.logical_and(gn < n_groups, g >= 1))
            def _():
                wait_store(bn)

            @pl.when(gn < n_groups)
            def _():
                issue_gathers(gn, bn)

            return carry

        lax.fori_loop(0, n_groups, group_body, 0)

        for t in range(n_groups - NBUF, n_groups):
            wait_store(t % NBUF)

    return scb


def kernel(x, table):
    V, D = table.shape
    xt = x.astype(jnp.int32).T
    tab_p = _tca(table.T)
    out_p = _make_scb(V)(xt, tab_p)
    out_t = _tcc(out_p)
    return out_t.transpose(2, 0, 1)


# speedup vs baseline: 3.0992x; 1.3294x over previous
import functools

import jax
import jax.numpy as jnp
from jax import lax
from jax.experimental import pallas as pl
from jax.experimental.pallas import tpu as pltpu
from jax.experimental.pallas import tpu_sc as plsc

BATCH = 16384
HIST = 50
EMB = 32
ROW = 128
OW = 64


def _tca_tab(table_t):
    V = table_t.shape[1]
    TBLK = 8192
    grid = (pl.cdiv(V, TBLK),)

    def body(tin, tout):
        t = tin[...].T.reshape(TBLK // 4, 4, EMB)
        for j in range(4):
            tout[:, EMB * j:EMB * (j + 1)] = t[:, j, :]

    return pl.pallas_call(
        body,
        grid=grid,
        in_specs=[pl.BlockSpec((EMB, TBLK), lambda i: (0, i))],
        out_specs=pl.BlockSpec((TBLK // 4, ROW), lambda i: (i, 0)),
        out_shape=jax.ShapeDtypeStruct((V // 4, ROW), jnp.float32),
    )(table_t)


def _tca_idx(xt):
    XBLK = 4096
    grid = (BATCH // XBLK,)

    def body(tin, tout):
        tout[:, :HIST] = tin[...].T

    return pl.pallas_call(
        body,
        grid=grid,
        in_specs=[pl.BlockSpec((HIST, XBLK), lambda i: (0, i))],
        out_specs=pl.BlockSpec((XBLK, ROW), lambda i: (i, 0)),
        out_shape=jax.ShapeDtypeStruct((BATCH, ROW), jnp.int32),
    )(xt)


def _tcc(outp):
    BB = 512
    RPB = HIST * OW // ROW
    grid = (BATCH // BB,)

    def body(tin, tout):
        t = tin[...].reshape(BB, HIST * OW).T
        tout[...] = t.reshape(HIST, OW, BB)[:, :EMB, :]

    return pl.pallas_call(
        body,
        grid=grid,
        in_specs=[pl.BlockSpec((BB * RPB, ROW), lambda i: (i, 0))],
        out_specs=pl.BlockSpec((HIST, EMB, BB), lambda i: (0, 0, i)),
        out_shape=jax.ShapeDtypeStruct((HIST, EMB, BATCH), jnp.float32),
    )(outp)


def _make_scb(V):
    info = plsc.get_sparse_core_info()
    NC, NS = info.num_cores, info.num_subcores
    NW = NC * NS
    b_per_w = BATCH // NW
    XST = 2
    xb_st = b_per_w // XST
    n_st = xb_st * HIST
    CH = 128
    GC = 4
    GROUP = CH * GC
    NBUF = 2
    n_groups = n_st // GROUP

    mesh = plsc.VectorSubcoreMesh(core_axis_name="c", subcore_axis_name="s")

    @functools.partial(
        pl.kernel,
        mesh=mesh,
        compiler_params=pltpu.CompilerParams(
            use_tc_tiling_on_sc=False, needs_layout_passes=False),
        out_type=jax.ShapeDtypeStruct((BATCH * HIST, OW), jnp.float32),
        scratch_types=[
            pltpu.VMEM((xb_st, ROW), jnp.int32),
            pltpu.VMEM((n_st,), jnp.int32),
            pltpu.VMEM((NBUF, GROUP, EMB), jnp.float32),
            pltpu.SemaphoreType.DMA((NBUF,)),
            pltpu.SemaphoreType.DMA((NBUF,)),
        ],
    )
    def scb(xb_hbm, tab_hbm, out_hbm, xv, xf, rows_v, gsem, ssem):
        cid = lax.axis_index("c")
        sid = lax.axis_index("s")
        wid = cid * NS + sid
        base = wid * b_per_w
        lanes = lax.iota(jnp.int32, 16)

        def x_stage(st, carry):
            pltpu.sync_copy(xb_hbm.at[pl.ds(base + st * xb_st, xb_st)], xv)

            def tr_body(k, carry2):
                s = lanes + k * 16
                vals = plsc.load_gather(xv, [s // HIST, s % HIST])
                xf[pl.ds(k * 16, 16)] = vals
                return carry2

            lax.fori_loop(0, n_st // 16, tr_body, 0)

            row0 = (base + st * xb_st) * HIST

            def issue_gathers(g, b):
                for j in range(GC):
                    pltpu.async_copy(
                        tab_hbm.at[xf.at[pl.ds(g * GROUP + j * CH, CH)]],
                        rows_v.at[b, pl.ds(j * CH, CH)],
                        gsem.at[b],
                    )

            def wait_gathers(g, b):
                for j in range(GC):
                    pltpu.make_async_copy(
                        tab_hbm.at[xf.at[pl.ds(g * GROUP + j * CH, CH)]],
                        rows_v.at[b, pl.ds(j * CH, CH)],
                        gsem.at[b],
                    ).wait()

            def wait_store(b):
                pltpu.make_async_copy(
                    rows_v.at[b],
                    out_hbm.at[pl.ds(0, GROUP), pl.ds(0, EMB)],
                    ssem.at[b],
                ).wait()

            issue_gathers(0, 0)

            def group_body(g, carry2):
                b = g % NBUF
                wait_gathers(g, b)
                pltpu.async_copy(
                    rows_v.at[b],
                    out_hbm.at[pl.ds(row0 + g * GROUP, GROUP),
                               pl.ds(0, EMB)],
                    ssem.at[b],
                )
                gn = g + 1
                bn = gn % NBUF

                @pl.when(jnp.logical_and(gn < n_groups, g >= 1))
                def _():
                    wait_store(bn)

                @pl.when(gn < n_groups)
                def _():
                    issue_gathers(gn, bn)

                return carry2

            lax.fori_loop(0, n_groups, group_body, 0)

            for t in range(n_groups - NBUF, n_groups):
                wait_store(t % NBUF)
            return carry

        lax.fori_loop(0, XST, x_stage, 0)

    return scb


def kernel(x, table):
    V, D = table.shape
    xt = x.astype(jnp.int32).T
    tab2d = _tca_tab(table.T)
    xb = _tca_idx(xt)
    tab_lin = tab2d.reshape(V, EMB)
    out64 = _make_scb(V)(xb, tab_lin)
    out_t = _tcc(out64.reshape(BATCH * HIST * OW // ROW, ROW))
    return out_t.transpose(2, 0, 1)
